# Initial kernel scaffold; baseline (speedup 1.0000x reference)
import jax
import jax.numpy as jnp
from jax.experimental import pallas as pl


def kernel(x_btI, w_ih, w_hh, b_ih, b_hh, w_fc, b_fc):
    raise NotImplementedError("write your pallas kernel here")


# speedup vs baseline: 2.0904x; 2.0904x over previous
import functools

import jax
import jax.numpy as jnp
from jax import lax
from jax.experimental import pallas as pl
from jax.experimental.pallas import tpu as pltpu


def _lstm_kernel(x_ref, wih_ref, whh_ref, b_ref, wfc_ref, bfc_ref, out_ref,
                 h_sc, c_sc, pre_sc, *, tc, bpc, h_dim):
    t_chunk = pl.program_id(1)

    @pl.when(t_chunk == 0)
    def _init():
        h_sc[...] = jnp.zeros_like(h_sc)
        c_sc[...] = jnp.zeros_like(c_sc)

    pre_sc[...] = (
        jnp.dot(x_ref[...], wih_ref[...], preferred_element_type=jnp.float32)
        + b_ref[...]
    )

    def step(t, carry):
        h, c = carry
        row0 = pl.multiple_of(t * bpc, bpc)
        gates = pre_sc[pl.ds(row0, bpc), :] + jnp.dot(
            h.astype(jnp.bfloat16), whh_ref[...],
            preferred_element_type=jnp.float32)
        i_g = jax.nn.sigmoid(gates[:, 0 * h_dim:1 * h_dim])
        f_g = jax.nn.sigmoid(gates[:, 1 * h_dim:2 * h_dim])
        g_g = jnp.tanh(gates[:, 2 * h_dim:3 * h_dim])
        o_g = jax.nn.sigmoid(gates[:, 3 * h_dim:4 * h_dim])
        c_new = f_g * c + i_g * g_g
        h_new = o_g * jnp.tanh(c_new)
        return h_new, c_new

    h_f, c_f = lax.fori_loop(0, tc, step, (h_sc[...], c_sc[...]), unroll=True)
    h_sc[...] = h_f
    c_sc[...] = c_f

    @pl.when(t_chunk == pl.num_programs(1) - 1)
    def _finalize():
        out_ref[...] = (
            jnp.dot(h_f.astype(jnp.bfloat16), wfc_ref[...],
                    preferred_element_type=jnp.float32)
            + bfc_ref[...]
        )


def kernel(x_btI, w_ih, w_hh, b_ih, b_hh, w_fc, b_fc):
    B, T, I = x_btI.shape
    H = w_hh.shape[1]
    O = w_fc.shape[0]
    f32 = jnp.float32
    bf16 = jnp.bfloat16

    ncores = 2
    bpc = B // ncores

    tc = 1
    for cand in range(1, min(T, 16) + 1):
        if T % cand == 0:
            tc = cand
    nc = T // tc

    x_tm = jnp.transpose(x_btI, (1, 0, 2)).reshape(T, ncores, bpc, I)
    x3 = jnp.transpose(x_tm, (1, 0, 2, 3)).reshape(ncores, T * bpc, I)
    x3 = x3.astype(bf16)

    wih = jnp.transpose(w_ih).astype(bf16)
    whh = jnp.transpose(w_hh).astype(bf16)
    b = (b_ih + b_hh).astype(f32).reshape(1, 4 * H)
    wfc = jnp.transpose(w_fc).astype(bf16)
    bfc = b_fc.astype(f32).reshape(1, O)

    grid_spec = pltpu.PrefetchScalarGridSpec(
        num_scalar_prefetch=0,
        grid=(ncores, nc),
        in_specs=[
            pl.BlockSpec((None, tc * bpc, I), lambda bi, ti: (bi, ti, 0)),
            pl.BlockSpec((I, 4 * H), lambda bi, ti: (0, 0)),
            pl.BlockSpec((H, 4 * H), lambda bi, ti: (0, 0)),
            pl.BlockSpec((1, 4 * H), lambda bi, ti: (0, 0)),
            pl.BlockSpec((H, O), lambda bi, ti: (0, 0)),
            pl.BlockSpec((1, O), lambda bi, ti: (0, 0)),
        ],
        out_specs=pl.BlockSpec((None, bpc, O), lambda bi, ti: (bi, 0, 0)),
        scratch_shapes=[
            pltpu.VMEM((bpc, H), f32),
            pltpu.VMEM((bpc, H), f32),
            pltpu.VMEM((tc * bpc, 4 * H), f32),
        ],
    )

    out = pl.pallas_call(
        functools.partial(_lstm_kernel, tc=tc, bpc=bpc, h_dim=H),
        out_shape=jax.ShapeDtypeStruct((ncores, bpc, O), f32),
        grid_spec=grid_spec,
        compiler_params=pltpu.CompilerParams(
            dimension_semantics=("parallel", "arbitrary"),
            vmem_limit_bytes=64 * 1024 * 1024,
        ),
    )(x3, wih, whh, b, wfc, bfc)

    return out.reshape(B, O)


# speedup vs baseline: 2.6177x; 1.2522x over previous
import functools

import jax
import jax.numpy as jnp
from jax.experimental import pallas as pl
from jax.experimental.pallas import tpu as pltpu


def _lstm_kernel(x_ref, w_ref, b_ref, wfc_ref, bfc_ref, out_ref,
                 h_sc, c_sc, *, tc, bpc, h_dim):
    t_chunk = pl.program_id(1)
    bf16 = jnp.bfloat16

    @pl.when(t_chunk == 0)
    def _init():
        h_sc[...] = jnp.zeros_like(h_sc)
        c_sc[...] = jnp.zeros_like(c_sc)

    h = h_sc[...]
    c = c_sc[...]
    for tl in range(tc):
        x_t = x_ref[:, tl, :].astype(bf16)
        inp = jnp.concatenate([h.astype(bf16), x_t], axis=1)
        gates = jnp.dot(inp, w_ref[...],
                        preferred_element_type=jnp.float32) + b_ref[...]
        i_g = jax.nn.sigmoid(gates[:, 0 * h_dim:1 * h_dim])
        f_g = jax.nn.sigmoid(gates[:, 1 * h_dim:2 * h_dim])
        g_g = jnp.tanh(gates[:, 2 * h_dim:3 * h_dim])
        o_g = jax.nn.sigmoid(gates[:, 3 * h_dim:4 * h_dim])
        c = f_g * c + i_g * g_g
        h = o_g * jnp.tanh(c)
    h_sc[...] = h
    c_sc[...] = c

    @pl.when(t_chunk == pl.num_programs(1) - 1)
    def _finalize():
        out_ref[...] = (
            jnp.dot(h.astype(bf16), wfc_ref[...],
                    preferred_element_type=jnp.float32)
            + bfc_ref[...]
        )


def kernel(x_btI, w_ih, w_hh, b_ih, b_hh, w_fc, b_fc):
    B, T, I = x_btI.shape
    H = w_hh.shape[1]
    O = w_fc.shape[0]
    f32 = jnp.float32
    bf16 = jnp.bfloat16

    ncores = 2
    bpc = B // ncores

    tc = 1
    for cand in range(1, min(T, 16) + 1):
        if T % cand == 0:
            tc = cand
    nc = T // tc

    x4 = x_btI.reshape(ncores, bpc, T, I)

    w = jnp.concatenate(
        [jnp.transpose(w_hh), jnp.transpose(w_ih)], axis=0).astype(bf16)
    b = (b_ih + b_hh).astype(f32).reshape(1, 4 * H)
    wfc = jnp.transpose(w_fc).astype(bf16)
    bfc = b_fc.astype(f32).reshape(1, O)

    grid_spec = pltpu.PrefetchScalarGridSpec(
        num_scalar_prefetch=0,
        grid=(ncores, nc),
        in_specs=[
            pl.BlockSpec((None, bpc, tc, I), lambda bi, ti: (bi, 0, ti, 0)),
            pl.BlockSpec((H + I, 4 * H), lambda bi, ti: (0, 0)),
            pl.BlockSpec((1, 4 * H), lambda bi, ti: (0, 0)),
            pl.BlockSpec((H, O), lambda bi, ti: (0, 0)),
            pl.BlockSpec((1, O), lambda bi, ti: (0, 0)),
        ],
        out_specs=pl.BlockSpec((None, bpc, O), lambda bi, ti: (bi, 0, 0)),
        scratch_shapes=[
            pltpu.VMEM((bpc, H), f32),
            pltpu.VMEM((bpc, H), f32),
        ],
    )

    out = pl.pallas_call(
        functools.partial(_lstm_kernel, tc=tc, bpc=bpc, h_dim=H),
        out_shape=jax.ShapeDtypeStruct((ncores, bpc, O), f32),
        grid_spec=grid_spec,
        compiler_params=pltpu.CompilerParams(
            dimension_semantics=("parallel", "arbitrary"),
            vmem_limit_bytes=64 * 1024 * 1024,
        ),
    )(x4, w, b, wfc, bfc)

    return out.reshape(B, O)


# speedup vs baseline: 2.6437x; 1.0099x over previous
import functools

import jax
import jax.numpy as jnp
from jax.experimental import pallas as pl
from jax.experimental.pallas import tpu as pltpu


def _sig(v):
    return 0.5 * jnp.tanh(0.5 * v) + 0.5


def _lstm_kernel(x_ref, wih_ref, whh_ref, b_ref, wfc_ref, bfc_ref, out_ref,
                 h_sc, c_sc, xt_sc, pre_sc, *, tc, bpc, h_dim, grp):
    t_chunk = pl.program_id(1)
    bf16 = jnp.bfloat16
    n_grp = bpc // grp

    @pl.when(t_chunk == 0)
    def _init():
        h_sc[...] = jnp.zeros_like(h_sc)
        c_sc[...] = jnp.zeros_like(c_sc)

    for tl in range(tc):
        xt_sc[tl * bpc:(tl + 1) * bpc, :] = x_ref[:, tl, :].astype(bf16)

    pre_sc[...] = (
        jnp.dot(xt_sc[...], wih_ref[...], preferred_element_type=jnp.float32)
        + b_ref[...]
    )

    hs = [h_sc[g * grp:(g + 1) * grp, :] for g in range(n_grp)]
    cs = [c_sc[g * grp:(g + 1) * grp, :] for g in range(n_grp)]
    for tl in range(tc):
        row0 = tl * bpc
        for g in range(n_grp):
            gates = pre_sc[row0 + g * grp:row0 + (g + 1) * grp, :] + jnp.dot(
                hs[g].astype(bf16), whh_ref[...],
                preferred_element_type=jnp.float32)
            i_g = _sig(gates[:, 0 * h_dim:1 * h_dim])
            f_g = _sig(gates[:, 1 * h_dim:2 * h_dim])
            g_g = jnp.tanh(gates[:, 2 * h_dim:3 * h_dim])
            o_g = _sig(gates[:, 3 * h_dim:4 * h_dim])
            cs[g] = f_g * cs[g] + i_g * g_g
            hs[g] = o_g * jnp.tanh(cs[g])

    for g in range(n_grp):
        h_sc[g * grp:(g + 1) * grp, :] = hs[g]
        c_sc[g * grp:(g + 1) * grp, :] = cs[g]

    @pl.when(t_chunk == pl.num_programs(1) - 1)
    def _finalize():
        for g in range(n_grp):
            out_ref[g * grp:(g + 1) * grp, :] = (
                jnp.dot(hs[g].astype(bf16), wfc_ref[...],
                        preferred_element_type=jnp.float32)
                + bfc_ref[...]
            )


def kernel(x_btI, w_ih, w_hh, b_ih, b_hh, w_fc, b_fc):
    B, T, I = x_btI.shape
    H = w_hh.shape[1]
    O = w_fc.shape[0]
    f32 = jnp.float32
    bf16 = jnp.bfloat16

    ncores = 2
    bpc = B // ncores
    grp = bpc // 2 if bpc % 16 == 0 else bpc

    tc = 1
    for cand in range(1, min(T, 16) + 1):
        if T % cand == 0:
            tc = cand
    nc = T // tc

    x4 = x_btI.reshape(ncores, bpc, T, I)

    wih = jnp.transpose(w_ih).astype(bf16)
    whh = jnp.transpose(w_hh).astype(bf16)
    b = (b_ih + b_hh).astype(f32).reshape(1, 4 * H)
    wfc = jnp.transpose(w_fc).astype(bf16)
    bfc = b_fc.astype(f32).reshape(1, O)

    grid_spec = pltpu.PrefetchScalarGridSpec(
        num_scalar_prefetch=0,
        grid=(ncores, nc),
        in_specs=[
            pl.BlockSpec((None, bpc, tc, I), lambda bi, ti: (bi, 0, ti, 0)),
            pl.BlockSpec((I, 4 * H), lambda bi, ti: (0, 0)),
            pl.BlockSpec((H, 4 * H), lambda bi, ti: (0, 0)),
            pl.BlockSpec((1, 4 * H), lambda bi, ti: (0, 0)),
            pl.BlockSpec((H, O), lambda bi, ti: (0, 0)),
            pl.BlockSpec((1, O), lambda bi, ti: (0, 0)),
        ],
        out_specs=pl.BlockSpec((None, bpc, O), lambda bi, ti: (bi, 0, 0)),
        scratch_shapes=[
            pltpu.VMEM((bpc, H), f32),
            pltpu.VMEM((bpc, H), f32),
            pltpu.VMEM((tc * bpc, I), bf16),
            pltpu.VMEM((tc * bpc, 4 * H), f32),
        ],
    )

    out = pl.pallas_call(
        functools.partial(_lstm_kernel, tc=tc, bpc=bpc, h_dim=H, grp=grp),
        out_shape=jax.ShapeDtypeStruct((ncores, bpc, O), f32),
        grid_spec=grid_spec,
        compiler_params=pltpu.CompilerParams(
            dimension_semantics=("parallel", "arbitrary"),
            vmem_limit_bytes=64 * 1024 * 1024,
        ),
    )(x4, wih, whh, b, wfc, bfc)

    return out.reshape(B, O)


# speedup vs baseline: 3.3817x; 1.2791x over previous
import functools

import jax
import jax.numpy as jnp
from jax.experimental import pallas as pl
from jax.experimental.pallas import tpu as pltpu


def _sig(v):
    return 0.5 * jnp.tanh(0.5 * v) + 0.5


def _lstm_kernel(x_ref, wih_ref, whh_ref, b_ref, wfc_ref, bfc_ref, out_ref,
                 h_sc, c_sc, xt_sc, pre_sc, *, tc, bp, h_dim, grp):
    t_chunk = pl.program_id(0)
    bf16 = jnp.bfloat16
    n_grp = bp // grp

    @pl.when(t_chunk == 0)
    def _init():
        h_sc[...] = jnp.zeros_like(h_sc)
        c_sc[...] = jnp.zeros_like(c_sc)

    for tl in range(tc):
        xt_sc[tl * bp:(tl + 1) * bp, :] = x_ref[:, tl, :].astype(bf16)

    pre_sc[...] = (
        jnp.dot(xt_sc[...], wih_ref[...], preferred_element_type=jnp.float32)
        + b_ref[...]
    )

    hs = [h_sc[g * grp:(g + 1) * grp, :] for g in range(n_grp)]
    cs = [c_sc[g * grp:(g + 1) * grp, :] for g in range(n_grp)]
    for tl in range(tc):
        row0 = tl * bp
        for g in range(n_grp):
            gates = pre_sc[row0 + g * grp:row0 + (g + 1) * grp, :] + jnp.dot(
                hs[g].astype(bf16), whh_ref[...],
                preferred_element_type=jnp.float32)
            i_g = _sig(gates[:, 0 * h_dim:1 * h_dim])
            f_g = _sig(gates[:, 1 * h_dim:2 * h_dim])
            g_g = jnp.tanh(gates[:, 2 * h_dim:3 * h_dim])
            o_g = _sig(gates[:, 3 * h_dim:4 * h_dim])
            cs[g] = f_g * cs[g] + i_g * g_g
            hs[g] = o_g * jnp.tanh(cs[g])

    for g in range(n_grp):
        h_sc[g * grp:(g + 1) * grp, :] = hs[g]
        c_sc[g * grp:(g + 1) * grp, :] = cs[g]

    @pl.when(t_chunk == pl.num_programs(0) - 1)
    def _finalize():
        for g in range(n_grp):
            out_ref[g * grp:(g + 1) * grp, :] = (
                jnp.dot(hs[g].astype(bf16), wfc_ref[...],
                        preferred_element_type=jnp.float32)
                + bfc_ref[...]
            )


def kernel(x_btI, w_ih, w_hh, b_ih, b_hh, w_fc, b_fc):
    B, T, I = x_btI.shape
    H = w_hh.shape[1]
    O = w_fc.shape[0]
    f32 = jnp.float32
    bf16 = jnp.bfloat16

    grp = B // 2 if B % 16 == 0 else B

    tc = 1
    for cand in range(1, min(T, 16) + 1):
        if T % cand == 0:
            tc = cand
    nc = T // tc

    wih = jnp.transpose(w_ih).astype(bf16)
    whh = jnp.transpose(w_hh).astype(bf16)
    b = (b_ih + b_hh).astype(f32).reshape(1, 4 * H)
    wfc = jnp.transpose(w_fc).astype(bf16)
    bfc = b_fc.astype(f32).reshape(1, O)

    grid_spec = pltpu.PrefetchScalarGridSpec(
        num_scalar_prefetch=0,
        grid=(nc,),
        in_specs=[
            pl.BlockSpec((B, tc, I), lambda ti: (0, ti, 0)),
            pl.BlockSpec((I, 4 * H), lambda ti: (0, 0)),
            pl.BlockSpec((H, 4 * H), lambda ti: (0, 0)),
            pl.BlockSpec((1, 4 * H), lambda ti: (0, 0)),
            pl.BlockSpec((H, O), lambda ti: (0, 0)),
            pl.BlockSpec((1, O), lambda ti: (0, 0)),
        ],
        out_specs=pl.BlockSpec((B, O), lambda ti: (0, 0)),
        scratch_shapes=[
            pltpu.VMEM((B, H), f32),
            pltpu.VMEM((B, H), f32),
            pltpu.VMEM((tc * B, I), bf16),
            pltpu.VMEM((tc * B, 4 * H), f32),
        ],
    )

    out = pl.pallas_call(
        functools.partial(_lstm_kernel, tc=tc, bp=B, h_dim=H, grp=grp),
        out_shape=jax.ShapeDtypeStruct((B, O), f32),
        grid_spec=grid_spec,
        compiler_params=pltpu.CompilerParams(
            dimension_semantics=("arbitrary",),
            vmem_limit_bytes=56 * 1024 * 1024,
        ),
    )(x_btI.reshape(B, T, I), wih, whh, b, wfc, bfc)

    return out
